# baseline (speedup 1.0000x reference)
import jax
import jax.numpy as jnp
from jax import lax
from jax.experimental import pallas as pl
from jax.experimental.pallas import tpu as pltpu
from jax.experimental.pallas import tpu_sc as plsc

N = 10000
NPAD = 10240
E = 160000
EPAD = 163840
F = 256
H = 256
C = 64
R = 8
G = 64
NREL = 9
TAB = 81920
RB = 128
ECH = EPAD // 128
NCH1 = ECH // 16
NCH2 = ECH // 32
BN = 512
_SC_MESH = dict(core_axis_name="c", subcore_axis_name="s")
_SC_PARAMS = pltpu.CompilerParams(needs_layout_passes=False)
_SC_PARAMS2 = pltpu.CompilerParams(needs_layout_passes=False,
                                   use_tc_tiling_on_sc=False)


BNA = 2048
BNB = 1024


def _tc_transform1_body(x_ref, w_ref, oa_ref, ob_ref):
    acc = jnp.dot(x_ref[...], w_ref[0], preferred_element_type=jnp.float32)
    oa_ref[...] = acc[:, :128]
    ob_ref[...] = acc[:, 128:]


def _tc_transform1(xpad16, w1cat16):
    nb = NPAD // BNA
    return pl.pallas_call(
        _tc_transform1_body,
        grid=(nb, NREL),
        in_specs=[
            pl.BlockSpec((BNA, F), lambda n, r: (n, 0)),
            pl.BlockSpec((1, F, H), lambda n, r: (r, 0, 0)),
        ],
        out_specs=[
            pl.BlockSpec((BNA, 128), lambda n, r: (r * nb + n, 0)),
            pl.BlockSpec((BNA, 128), lambda n, r: (r * nb + n, 0)),
        ],
        out_shape=[
            jax.ShapeDtypeStruct((NREL * NPAD, 128), jnp.float32),
            jax.ShapeDtypeStruct((NREL * NPAD, 128), jnp.float32),
        ],
    )(xpad16, w1cat16)


def _sc0_body(comp2, rcp, cnt_sh, zcnt, compb, onesb, sema):
    c = lax.axis_index("c")
    s = lax.axis_index("s")
    zlen = TAB // 16

    def _zl(i, _):
        zcnt[pl.ds(i * 16, 16)] = jnp.zeros((16,), jnp.float32)
        return 0
    lax.fori_loop(0, zlen // 16, _zl, 0)

    def _ol(i, _):
        onesb[pl.ds(i * 16, 16)] = jnp.ones((16,), jnp.float32)
        return 0
    lax.fori_loop(0, RB // 16, _ol, 0)

    pltpu.sync_copy(zcnt, cnt_sh.at[pl.ds(s * zlen, zlen)])
    plsc.subcore_barrier()

    pltpu.sync_copy(comp2.at[pl.ds(s * NCH1, NCH1)], compb)
    def _fire(ch, _):
        pltpu.async_copy(onesb, cnt_sh.at[compb.at[ch]], sema, add=True)
        return 0
    lax.fori_loop(0, NCH1, _fire, 0)
    def _drain(ch, _):
        pltpu.make_async_copy(onesb, cnt_sh.at[compb.at[0]], sema).wait()
        return 0
    lax.fori_loop(0, NCH1, _drain, 0)
    plsc.subcore_barrier()

    pltpu.sync_copy(cnt_sh.at[pl.ds(s * zlen, zlen)], zcnt)
    def _recip(i, _):
        v = zcnt[pl.ds(i * 16, 16)]
        zcnt[pl.ds(i * 16, 16)] = 1.0 / jnp.maximum(v, 1.0)
        return 0
    lax.fori_loop(0, zlen // 16, _recip, 0)
    @pl.when(c == 0)
    def _():
        pltpu.sync_copy(zcnt, rcp.at[pl.ds(s * zlen, zlen)])


def _sc_count(comp2):
    return pl.kernel(
        _sc0_body,
        out_type=[jax.ShapeDtypeStruct((TAB,), jnp.float32)],
        mesh=plsc.VectorSubcoreMesh(**_SC_MESH),
        compiler_params=_SC_PARAMS,
        scratch_types=[
            pltpu.VMEM_SHARED((TAB,), jnp.float32),
            pltpu.VMEM((TAB // 16,), jnp.float32),
            pltpu.VMEM((NCH1, RB), jnp.int32),
            pltpu.VMEM((RB,), jnp.float32),
            pltpu.SemaphoreType.DMA,
        ],
    )(comp2)


def _sc1_body(xwA, xwB, combo2, rcp, aggA, aggB,
              acc_sh, comboall, rowq, gixq, dstq, compq, normq,
              sg0, sg1, sn0, sn1, ss0, ss1):
    c = lax.axis_index("c")
    s = lax.axis_index("s")
    sg = (sg0, sg1)
    sn = (sn0, sn1)
    ss = (ss0, ss1)

    def _zrow(i, _):
        for k in range(8):
            rowq[0, i, pl.ds(k * 16, 16)] = jnp.zeros((16,), jnp.float32)
        return 0
    lax.fori_loop(0, RB, _zrow, 0)
    off = 0
    for sz in (128, 128, 128, 128, 112):
        pltpu.sync_copy(rowq.at[0].at[pl.ds(0, sz)],
                        acc_sh.at[pl.ds(s * 624 + off, sz)])
        off += sz
    @pl.when(s == 0)
    def _():
        pltpu.sync_copy(rowq.at[0].at[pl.ds(0, 16)], acc_sh.at[pl.ds(9984, 16)])
    plsc.subcore_barrier()

    pltpu.sync_copy(combo2.at[pl.ds(s * NCH1, NCH1)], comboall)

    def _derive(slot, ch):
        for g in range(8):
            v = comboall[ch, pl.ds(g * 16, 16)]
            d = v & 16383
            es = lax.shift_right_logical(v, 14)
            sr = es & 16383
            et = lax.shift_right_logical(es, 14)
            dstq[slot, pl.ds(g * 16, 16)] = d
            gixq[slot, pl.ds(g * 16, 16)] = et * NPAD + sr
            compq[slot, pl.ds(g * 16, 16)] = d * R + et

    def _fire(slot):
        pltpu.async_copy(rcp.at[compq.at[slot]], normq.at[slot], sn[slot])
        @pl.when(c == 0)
        def _():
            pltpu.async_copy(xwA.at[gixq.at[slot]], rowq.at[slot], sg[slot])
        @pl.when(c == 1)
        def _():
            pltpu.async_copy(xwB.at[gixq.at[slot]], rowq.at[slot], sg[slot])

    for slot in (0, 1):
        _derive(slot, jnp.int32(slot))
        _fire(slot)

    def _proc(slot):
        pltpu.make_async_copy(xwA.at[gixq.at[slot]], rowq.at[slot],
                              sg[slot]).wait()
        pltpu.make_async_copy(rcp.at[compq.at[slot]], normq.at[slot],
                              sn[slot]).wait()
        def _scale(jj, _):
            j = 2 * jj
            for u in range(2):
                nj = plsc.load_gather(
                    normq, [jnp.full((16,), slot, jnp.int32),
                            jnp.full((16,), j + u, jnp.int32)])
                for k in range(8):
                    rowq[slot, j + u, pl.ds(k * 16, 16)] = (
                        rowq[slot, j + u, pl.ds(k * 16, 16)] * nj)
            return 0
        lax.fori_loop(0, RB // 2, _scale, 0)
        pltpu.async_copy(rowq.at[slot], acc_sh.at[dstq.at[slot]], ss[slot],
                         add=True)

    def _sdrain(slot):
        pltpu.make_async_copy(rowq.at[slot], acc_sh.at[dstq.at[slot]],
                              ss[slot]).wait()

    def _pair(i, _):
        _proc(0)
        @pl.when(i > 0)
        def _():
            _sdrain(1)
        @pl.when(2 * i + 3 < NCH1)
        def _():
            _derive(1, 2 * i + 3)
            _fire(1)
        _proc(1)
        _sdrain(0)
        @pl.when(2 * i + 2 < NCH1)
        def _():
            _derive(0, 2 * i + 2)
            _fire(0)
        return 0
    lax.fori_loop(0, NCH1 // 2, _pair, 0)
    _sdrain(1)
    plsc.subcore_barrier()

    base = s * 624
    @pl.when(c == 0)
    def _():
        pltpu.sync_copy(acc_sh.at[pl.ds(base, 624)], aggA.at[pl.ds(base, 624)])
        @pl.when(s == 0)
        def _():
            pltpu.sync_copy(acc_sh.at[pl.ds(9984, 16)],
                            aggA.at[pl.ds(9984, 16)])
    @pl.when(c == 1)
    def _():
        pltpu.sync_copy(acc_sh.at[pl.ds(base, 624)], aggB.at[pl.ds(base, 624)])
        @pl.when(s == 0)
        def _():
            pltpu.sync_copy(acc_sh.at[pl.ds(9984, 16)],
                            aggB.at[pl.ds(9984, 16)])


def _sc_agg1(xwA_f, xwB_f, combo2, rcp):
    return pl.kernel(
        _sc1_body,
        out_type=[
            jax.ShapeDtypeStruct((NPAD, 128), jnp.float32),
            jax.ShapeDtypeStruct((NPAD, 128), jnp.float32),
        ],
        mesh=plsc.VectorSubcoreMesh(**_SC_MESH),
        compiler_params=_SC_PARAMS,
        scratch_types=[
            pltpu.VMEM_SHARED((N, 128), jnp.float32),
            pltpu.VMEM((NCH1, RB), jnp.int32),
            pltpu.VMEM((2, RB, 128), jnp.float32),
            pltpu.VMEM((2, RB), jnp.int32),
            pltpu.VMEM((2, RB), jnp.int32),
            pltpu.VMEM((2, RB), jnp.int32),
            pltpu.VMEM((2, RB), jnp.float32),
            pltpu.SemaphoreType.DMA,
            pltpu.SemaphoreType.DMA,
            pltpu.SemaphoreType.DMA,
            pltpu.SemaphoreType.DMA,
            pltpu.SemaphoreType.DMA,
            pltpu.SemaphoreType.DMA,
        ],
    )(xwA_f, xwB_f, combo2, rcp)


def _tc_b_body(aggA_ref, aggB_ref, rootA_ref, rootB_ref, b1_ref, w2_ref,
               out_ref, hA_s, hB_s):
    n = pl.program_id(0)
    r = pl.program_id(1)
    @pl.when(r == 0)
    def _():
        rid = n * BNB + lax.broadcasted_iota(jnp.int32, (BNB, 1), 0)
        valid = rid < N
        hA = jnp.maximum(aggA_ref[...] + rootA_ref[...] + b1_ref[0, :128], 0.0)
        hB = jnp.maximum(aggB_ref[...] + rootB_ref[...] + b1_ref[0, 128:], 0.0)
        hA_s[...] = jnp.where(valid, hA, 0.0).astype(jnp.bfloat16)
        hB_s[...] = jnp.where(valid, hB, 0.0).astype(jnp.bfloat16)
    out_ref[...] = (
        jnp.dot(hA_s[...], w2_ref[0, :128, :],
                preferred_element_type=jnp.float32)
        + jnp.dot(hB_s[...], w2_ref[0, 128:, :],
                  preferred_element_type=jnp.float32))


def _tc_b(aggA, aggB, rootA, rootB, b1r, w2cat16):
    nb = NPAD // BNB
    return pl.pallas_call(
        _tc_b_body,
        grid=(nb, NREL),
        in_specs=[
            pl.BlockSpec((BNB, 128), lambda n, r: (n, 0)),
            pl.BlockSpec((BNB, 128), lambda n, r: (n, 0)),
            pl.BlockSpec((BNB, 128), lambda n, r: (n, 0)),
            pl.BlockSpec((BNB, 128), lambda n, r: (n, 0)),
            pl.BlockSpec((1, H), lambda n, r: (0, 0)),
            pl.BlockSpec((1, H, C), lambda n, r: (r, 0, 0)),
        ],
        out_specs=pl.BlockSpec((BNB, C), lambda n, r: (r * nb + n, 0)),
        out_shape=jax.ShapeDtypeStruct((NREL * NPAD, C), jnp.float32),
        scratch_shapes=[pltpu.VMEM((BNB, 128), jnp.bfloat16),
                        pltpu.VMEM((BNB, 128), jnp.bfloat16)],
    )(aggA, aggB, rootA, rootB, b1r, w2cat16)


def _sc2_body(xw2, combo2, rcp, pA, pB,
              acc_sh, comboall, rowq, gixq, dstq, compq, normq,
              sg0, sg1, sn0, sn1, ss0, ss1):
    c = lax.axis_index("c")
    s = lax.axis_index("s")
    sg = (sg0, sg1)
    sn = (sn0, sn1)
    ss = (ss0, ss1)
    w = c * 16 + s

    def _zrow(i, _):
        for k in range(4):
            rowq[0, i, pl.ds(k * 16, 16)] = jnp.zeros((16,), jnp.float32)
        return 0
    lax.fori_loop(0, RB, _zrow, 0)
    off = 0
    for sz in (128, 128, 128, 128, 112):
        pltpu.sync_copy(rowq.at[0].at[pl.ds(0, sz)],
                        acc_sh.at[pl.ds(s * 624 + off, sz)])
        off += sz
    @pl.when(s == 0)
    def _():
        pltpu.sync_copy(rowq.at[0].at[pl.ds(0, 16)], acc_sh.at[pl.ds(9984, 16)])
    plsc.subcore_barrier()

    pltpu.sync_copy(combo2.at[pl.ds(w * NCH2, NCH2)], comboall)

    def _derive(slot, ch):
        for g in range(8):
            v = comboall[ch, pl.ds(g * 16, 16)]
            d = v & 16383
            es = lax.shift_right_logical(v, 14)
            sr = es & 16383
            et = lax.shift_right_logical(es, 14)
            dstq[slot, pl.ds(g * 16, 16)] = d
            gixq[slot, pl.ds(g * 16, 16)] = et * NPAD + sr
            compq[slot, pl.ds(g * 16, 16)] = d * R + et

    def _fire(slot):
        pltpu.async_copy(rcp.at[compq.at[slot]], normq.at[slot], sn[slot])
        pltpu.async_copy(xw2.at[gixq.at[slot]], rowq.at[slot], sg[slot])

    for slot in (0, 1):
        _derive(slot, jnp.int32(slot))
        _fire(slot)

    def _proc(slot):
        pltpu.make_async_copy(xw2.at[gixq.at[slot]], rowq.at[slot],
                              sg[slot]).wait()
        pltpu.make_async_copy(rcp.at[compq.at[slot]], normq.at[slot],
                              sn[slot]).wait()
        def _scale(jj, _):
            j = 4 * jj
            for u in range(4):
                nj = plsc.load_gather(
                    normq, [jnp.full((16,), slot, jnp.int32),
                            jnp.full((16,), j + u, jnp.int32)])
                for k in range(4):
                    rowq[slot, j + u, pl.ds(k * 16, 16)] = (
                        rowq[slot, j + u, pl.ds(k * 16, 16)] * nj)
            return 0
        lax.fori_loop(0, RB // 4, _scale, 0)
        pltpu.async_copy(rowq.at[slot], acc_sh.at[dstq.at[slot]], ss[slot],
                         add=True)

    def _sdrain(slot):
        pltpu.make_async_copy(rowq.at[slot], acc_sh.at[dstq.at[slot]],
                              ss[slot]).wait()

    def _pair(i, _):
        _proc(0)
        @pl.when(i > 0)
        def _():
            _sdrain(1)
        @pl.when(2 * i + 3 < NCH2)
        def _():
            _derive(1, 2 * i + 3)
            _fire(1)
        _proc(1)
        _sdrain(0)
        @pl.when(2 * i + 2 < NCH2)
        def _():
            _derive(0, 2 * i + 2)
            _fire(0)
        return 0
    lax.fori_loop(0, NCH2 // 2, _pair, 0)
    _sdrain(1)
    plsc.subcore_barrier()

    base = s * 624
    @pl.when(c == 0)
    def _():
        pltpu.sync_copy(acc_sh.at[pl.ds(base, 624)], pA.at[pl.ds(base, 624)])
        @pl.when(s == 0)
        def _():
            pltpu.sync_copy(acc_sh.at[pl.ds(9984, 16)], pA.at[pl.ds(9984, 16)])
    @pl.when(c == 1)
    def _():
        pltpu.sync_copy(acc_sh.at[pl.ds(base, 624)], pB.at[pl.ds(base, 624)])
        @pl.when(s == 0)
        def _():
            pltpu.sync_copy(acc_sh.at[pl.ds(9984, 16)], pB.at[pl.ds(9984, 16)])


def _sc_agg2(xw2_f, combo2, rcp):
    return pl.kernel(
        _sc2_body,
        out_type=[
            jax.ShapeDtypeStruct((NPAD, C), jnp.float32),
            jax.ShapeDtypeStruct((NPAD, C), jnp.float32),
        ],
        mesh=plsc.VectorSubcoreMesh(**_SC_MESH),
        compiler_params=_SC_PARAMS2,
        scratch_types=[
            pltpu.VMEM_SHARED((N, C), jnp.float32),
            pltpu.VMEM((NCH2, RB), jnp.int32),
            pltpu.VMEM((2, RB, C), jnp.float32),
            pltpu.VMEM((2, RB), jnp.int32),
            pltpu.VMEM((2, RB), jnp.int32),
            pltpu.VMEM((2, RB), jnp.int32),
            pltpu.VMEM((2, RB), jnp.float32),
            pltpu.SemaphoreType.DMA,
            pltpu.SemaphoreType.DMA,
            pltpu.SemaphoreType.DMA,
            pltpu.SemaphoreType.DMA,
            pltpu.SemaphoreType.DMA,
            pltpu.SemaphoreType.DMA,
        ],
    )(xw2_f, combo2, rcp)


def _tc_c_body(p0_ref, p1_ref, root_ref, b2_ref, batch_ref,
               node_ref, graph_ref, acc_ref):
    i = pl.program_id(0)
    rid = i * BN + lax.broadcasted_iota(jnp.int32, (BN, 1), 0)
    nf = jnp.maximum(p0_ref[...] + p1_ref[...] + root_ref[...] + b2_ref[0],
                     0.0)
    nf = jnp.where(rid < N, nf, 0.0)
    m = jnp.max(nf, axis=1, keepdims=True)
    e = jnp.exp(nf - m)
    node_ref[...] = e / jnp.sum(e, axis=1, keepdims=True)

    bb = batch_ref[0]
    gids = lax.broadcasted_iota(jnp.int32, (G, BN), 0)
    oh = (bb == gids).astype(jnp.float32)
    ext = jnp.concatenate([nf, jnp.ones((BN, C), jnp.float32)], axis=1)
    contrib = jnp.dot(oh, ext, preferred_element_type=jnp.float32)
    @pl.when(i == 0)
    def _():
        acc_ref[...] = jnp.zeros_like(acc_ref)
    acc_ref[...] += contrib
    @pl.when(i == pl.num_programs(0) - 1)
    def _():
        rep = acc_ref[:, :C] / jnp.maximum(acc_ref[:, C:], 1.0)
        m2 = jnp.max(rep, axis=1, keepdims=True)
        e2 = jnp.exp(rep - m2)
        graph_ref[...] = e2 / jnp.sum(e2, axis=1, keepdims=True)


def _tc_c(pA, pB, root2, b2r, batch3d):
    return pl.pallas_call(
        _tc_c_body,
        grid=(NPAD // BN,),
        in_specs=[
            pl.BlockSpec((BN, C), lambda n: (n, 0)),
            pl.BlockSpec((BN, C), lambda n: (n, 0)),
            pl.BlockSpec((BN, C), lambda n: (n, 0)),
            pl.BlockSpec((1, C), lambda n: (0, 0)),
            pl.BlockSpec((1, 1, BN), lambda n: (n, 0, 0)),
        ],
        out_specs=[
            pl.BlockSpec((BN, C), lambda n: (n, 0)),
            pl.BlockSpec((G, G), lambda n: (0, 0)),
        ],
        out_shape=[
            jax.ShapeDtypeStruct((NPAD, C), jnp.float32),
            jax.ShapeDtypeStruct((G, G), jnp.float32),
        ],
        scratch_shapes=[pltpu.VMEM((G, 2 * C), jnp.float32)],
    )(pA, pB, root2, b2r, batch3d)


def kernel(x, edge_index, edge_type, batch, W1_rel, W1_root, b1,
           W2_rel, W2_root, b2):
    src = edge_index[0].astype(jnp.int32)
    dst = edge_index[1].astype(jnp.int32)
    et = edge_type.astype(jnp.int32)

    xpad = jnp.pad(x.astype(jnp.float32), ((0, NPAD - N), (0, 0)))
    w1cat = jnp.concatenate([W1_rel, W1_root[None]], axis=0)
    w2cat = jnp.concatenate([W2_rel, W2_root[None]], axis=0)

    pad_e = EPAD - E
    combo = (et * 16384 + src) * 16384 + dst
    combo2 = jnp.concatenate(
        [combo, jnp.full((pad_e,), N * 16384, jnp.int32)]).reshape(ECH, RB)
    comp2 = jnp.concatenate(
        [dst * R + et, jnp.full((pad_e,), N * R, jnp.int32)]).reshape(ECH, RB)
    batch3d = jnp.concatenate(
        [batch.astype(jnp.int32), jnp.full((NPAD - N,), G, jnp.int32)]
    ).reshape(NPAD // BN, 1, BN)

    rcp = _sc_count(comp2)[0]
    xwA, xwB = _tc_transform1(xpad.astype(jnp.bfloat16),
                              w1cat.astype(jnp.bfloat16))
    aggA, aggB = _sc_agg1(xwA, xwB, combo2, rcp)
    xw2 = _tc_b(aggA, aggB, xwA[8 * NPAD:], xwB[8 * NPAD:],
                b1.reshape(1, H), w2cat.astype(jnp.bfloat16))
    pA, pB = _sc_agg2(xw2, combo2, rcp)
    node_full, graph_out = _tc_c(pA, pB, xw2[8 * NPAD:], b2.reshape(1, C),
                                 batch3d)
    return (node_full[:N], graph_out)


# speedup vs baseline: 1.0664x; 1.0664x over previous
import jax
import jax.numpy as jnp
from jax import lax
from jax.experimental import pallas as pl
from jax.experimental.pallas import tpu as pltpu
from jax.experimental.pallas import tpu_sc as plsc

N = 10000
NPAD = 10240
E = 160000
EPAD = 163840
F = 256
H = 256
C = 64
R = 8
G = 64
NREL = 9
TAB = 81920
RB = 128
ECH = EPAD // 128
NCH1 = ECH // 16
NCH2 = ECH // 32
BN = 512
_SC_MESH = dict(core_axis_name="c", subcore_axis_name="s")
_SC_PARAMS = pltpu.CompilerParams(needs_layout_passes=False)
_SC_PARAMS2 = pltpu.CompilerParams(needs_layout_passes=False,
                                   use_tc_tiling_on_sc=False)


BNA = 2048
BNB = 1024


def _tc_transform1_body(x_ref, w_ref, oa_ref, ob_ref):
    acc = jnp.dot(x_ref[...], w_ref[0], preferred_element_type=jnp.float32)
    oa_ref[...] = acc[:, :128]
    ob_ref[...] = acc[:, 128:]


def _tc_transform1(xpad16, w1cat16):
    nb = NPAD // BNA
    return pl.pallas_call(
        _tc_transform1_body,
        grid=(nb, NREL),
        in_specs=[
            pl.BlockSpec((BNA, F), lambda n, r: (n, 0)),
            pl.BlockSpec((1, F, H), lambda n, r: (r, 0, 0)),
        ],
        out_specs=[
            pl.BlockSpec((BNA, 128), lambda n, r: (r * nb + n, 0)),
            pl.BlockSpec((BNA, 128), lambda n, r: (r * nb + n, 0)),
        ],
        out_shape=[
            jax.ShapeDtypeStruct((NREL * NPAD, 128), jnp.float32),
            jax.ShapeDtypeStruct((NREL * NPAD, 128), jnp.float32),
        ],
    )(xpad16, w1cat16)


def _sc0_body(comp2, rcp, cnt_sh, zcnt, compb, onesb, sema):
    c = lax.axis_index("c")
    s = lax.axis_index("s")
    zlen = TAB // 16

    def _zl(i, _):
        zcnt[pl.ds(i * 16, 16)] = jnp.zeros((16,), jnp.float32)
        return 0
    lax.fori_loop(0, zlen // 16, _zl, 0)

    def _ol(i, _):
        onesb[pl.ds(i * 16, 16)] = jnp.ones((16,), jnp.float32)
        return 0
    lax.fori_loop(0, RB // 16, _ol, 0)

    pltpu.sync_copy(zcnt, cnt_sh.at[pl.ds(s * zlen, zlen)])
    plsc.subcore_barrier()

    pltpu.sync_copy(comp2.at[pl.ds(s * NCH1, NCH1)], compb)
    def _fire(ch, _):
        pltpu.async_copy(onesb, cnt_sh.at[compb.at[ch]], sema, add=True)
        return 0
    lax.fori_loop(0, NCH1, _fire, 0)
    def _drain(ch, _):
        pltpu.make_async_copy(onesb, cnt_sh.at[compb.at[0]], sema).wait()
        return 0
    lax.fori_loop(0, NCH1, _drain, 0)
    plsc.subcore_barrier()

    pltpu.sync_copy(cnt_sh.at[pl.ds(s * zlen, zlen)], zcnt)
    def _recip(i, _):
        v = zcnt[pl.ds(i * 16, 16)]
        zcnt[pl.ds(i * 16, 16)] = 1.0 / jnp.maximum(v, 1.0)
        return 0
    lax.fori_loop(0, zlen // 16, _recip, 0)
    @pl.when(c == 0)
    def _():
        pltpu.sync_copy(zcnt, rcp.at[pl.ds(s * zlen, zlen)])


def _sc_count(comp2):
    return pl.kernel(
        _sc0_body,
        out_type=[jax.ShapeDtypeStruct((TAB,), jnp.float32)],
        mesh=plsc.VectorSubcoreMesh(**_SC_MESH),
        compiler_params=_SC_PARAMS,
        scratch_types=[
            pltpu.VMEM_SHARED((TAB,), jnp.float32),
            pltpu.VMEM((TAB // 16,), jnp.float32),
            pltpu.VMEM((NCH1, RB), jnp.int32),
            pltpu.VMEM((RB,), jnp.float32),
            pltpu.SemaphoreType.DMA,
        ],
    )(comp2)


def _sc1_body(xwA, xwB, combo2, rcp, aggA, aggB,
              acc_sh, comboall, rowq, gixq, dstq, compq, normq,
              sg0, sg1, sn0, sn1):
    c = lax.axis_index("c")
    s = lax.axis_index("s")
    sg = (sg0, sg1)
    sn = (sn0, sn1)

    def _zrow(i, _):
        for k in range(8):
            rowq[0, i, pl.ds(k * 16, 16)] = jnp.zeros((16,), jnp.float32)
        return 0
    lax.fori_loop(0, RB, _zrow, 0)
    off = 0
    for sz in (128, 128, 128, 128, 112):
        pltpu.sync_copy(rowq.at[0].at[pl.ds(0, sz)],
                        acc_sh.at[pl.ds(s * 624 + off, sz)])
        off += sz
    @pl.when(s == 0)
    def _():
        pltpu.sync_copy(rowq.at[0].at[pl.ds(0, 16)], acc_sh.at[pl.ds(9984, 16)])
    plsc.subcore_barrier()

    pltpu.sync_copy(combo2.at[pl.ds(s * NCH1, NCH1)], comboall)

    def _derive(slot, ch):
        for g in range(8):
            v = comboall[ch, pl.ds(g * 16, 16)]
            d = v & 16383
            es = lax.shift_right_logical(v, 14)
            sr = es & 16383
            et = lax.shift_right_logical(es, 14)
            dstq[slot, pl.ds(g * 16, 16)] = d
            gixq[slot, pl.ds(g * 16, 16)] = et * NPAD + sr
            compq[slot, pl.ds(g * 16, 16)] = d * R + et

    def _fire(slot):
        pltpu.async_copy(rcp.at[compq.at[slot]], normq.at[slot], sn[slot])
        @pl.when(c == 0)
        def _():
            pltpu.async_copy(xwA.at[gixq.at[slot]], rowq.at[slot], sg[slot])
        @pl.when(c == 1)
        def _():
            pltpu.async_copy(xwB.at[gixq.at[slot]], rowq.at[slot], sg[slot])

    for slot in (0, 1):
        _derive(slot, jnp.int32(slot))
        _fire(slot)

    def _pair(i, _):
        for slot in (0, 1):
            ch = 2 * i + slot
            pltpu.make_async_copy(xwA.at[gixq.at[slot]], rowq.at[slot],
                                  sg[slot]).wait()
            pltpu.make_async_copy(rcp.at[compq.at[slot]], normq.at[slot],
                                  sn[slot]).wait()
            def _scale(jj, _):
                j = 2 * jj
                for u in range(2):
                    nj = plsc.load_gather(
                        normq, [jnp.full((16,), slot, jnp.int32),
                                jnp.full((16,), j + u, jnp.int32)])
                    for k in range(8):
                        rowq[slot, j + u, pl.ds(k * 16, 16)] = (
                            rowq[slot, j + u, pl.ds(k * 16, 16)] * nj)
                return 0
            lax.fori_loop(0, RB // 2, _scale, 0)
            pltpu.sync_copy(rowq.at[slot], acc_sh.at[dstq.at[slot]], add=True)
            @pl.when(ch + 2 < NCH1)
            def _():
                _derive(slot, ch + 2)
                _fire(slot)
        return 0
    lax.fori_loop(0, NCH1 // 2, _pair, 0)
    plsc.subcore_barrier()

    base = s * 624
    @pl.when(c == 0)
    def _():
        pltpu.sync_copy(acc_sh.at[pl.ds(base, 624)], aggA.at[pl.ds(base, 624)])
        @pl.when(s == 0)
        def _():
            pltpu.sync_copy(acc_sh.at[pl.ds(9984, 16)],
                            aggA.at[pl.ds(9984, 16)])
    @pl.when(c == 1)
    def _():
        pltpu.sync_copy(acc_sh.at[pl.ds(base, 624)], aggB.at[pl.ds(base, 624)])
        @pl.when(s == 0)
        def _():
            pltpu.sync_copy(acc_sh.at[pl.ds(9984, 16)],
                            aggB.at[pl.ds(9984, 16)])


def _sc_agg1(xwA_f, xwB_f, combo2, rcp):
    return pl.kernel(
        _sc1_body,
        out_type=[
            jax.ShapeDtypeStruct((NPAD, 128), jnp.float32),
            jax.ShapeDtypeStruct((NPAD, 128), jnp.float32),
        ],
        mesh=plsc.VectorSubcoreMesh(**_SC_MESH),
        compiler_params=_SC_PARAMS,
        scratch_types=[
            pltpu.VMEM_SHARED((N, 128), jnp.float32),
            pltpu.VMEM((NCH1, RB), jnp.int32),
            pltpu.VMEM((2, RB, 128), jnp.float32),
            pltpu.VMEM((2, RB), jnp.int32),
            pltpu.VMEM((2, RB), jnp.int32),
            pltpu.VMEM((2, RB), jnp.int32),
            pltpu.VMEM((2, RB), jnp.float32),
            pltpu.SemaphoreType.DMA,
            pltpu.SemaphoreType.DMA,
            pltpu.SemaphoreType.DMA,
            pltpu.SemaphoreType.DMA,
        ],
    )(xwA_f, xwB_f, combo2, rcp)


def _tc_b_body(aggA_ref, aggB_ref, rootA_ref, rootB_ref, b1_ref, w2_ref,
               out_ref, hA_s, hB_s):
    n = pl.program_id(0)
    r = pl.program_id(1)
    @pl.when(r == 0)
    def _():
        rid = n * BNB + lax.broadcasted_iota(jnp.int32, (BNB, 1), 0)
        valid = rid < N
        hA = jnp.maximum(aggA_ref[...] + rootA_ref[...] + b1_ref[0, :128], 0.0)
        hB = jnp.maximum(aggB_ref[...] + rootB_ref[...] + b1_ref[0, 128:], 0.0)
        hA_s[...] = jnp.where(valid, hA, 0.0).astype(jnp.bfloat16)
        hB_s[...] = jnp.where(valid, hB, 0.0).astype(jnp.bfloat16)
    out_ref[...] = (
        jnp.dot(hA_s[...], w2_ref[0, :128, :],
                preferred_element_type=jnp.float32)
        + jnp.dot(hB_s[...], w2_ref[0, 128:, :],
                  preferred_element_type=jnp.float32))


def _tc_b(aggA, aggB, rootA, rootB, b1r, w2cat16):
    nb = NPAD // BNB
    return pl.pallas_call(
        _tc_b_body,
        grid=(nb, NREL),
        in_specs=[
            pl.BlockSpec((BNB, 128), lambda n, r: (n, 0)),
            pl.BlockSpec((BNB, 128), lambda n, r: (n, 0)),
            pl.BlockSpec((BNB, 128), lambda n, r: (n, 0)),
            pl.BlockSpec((BNB, 128), lambda n, r: (n, 0)),
            pl.BlockSpec((1, H), lambda n, r: (0, 0)),
            pl.BlockSpec((1, H, C), lambda n, r: (r, 0, 0)),
        ],
        out_specs=pl.BlockSpec((BNB, C), lambda n, r: (r * nb + n, 0)),
        out_shape=jax.ShapeDtypeStruct((NREL * NPAD, C), jnp.float32),
        scratch_shapes=[pltpu.VMEM((BNB, 128), jnp.bfloat16),
                        pltpu.VMEM((BNB, 128), jnp.bfloat16)],
    )(aggA, aggB, rootA, rootB, b1r, w2cat16)


def _sc2_body(xw2, combo2, rcp, pA, pB,
              acc_sh, comboall, rowq, gixq, dstq, compq, normq,
              sg0, sg1, sn0, sn1):
    c = lax.axis_index("c")
    s = lax.axis_index("s")
    sg = (sg0, sg1)
    sn = (sn0, sn1)
    w = c * 16 + s

    def _zrow(i, _):
        for k in range(4):
            rowq[0, i, pl.ds(k * 16, 16)] = jnp.zeros((16,), jnp.float32)
        return 0
    lax.fori_loop(0, RB, _zrow, 0)
    off = 0
    for sz in (128, 128, 128, 128, 112):
        pltpu.sync_copy(rowq.at[0].at[pl.ds(0, sz)],
                        acc_sh.at[pl.ds(s * 624 + off, sz)])
        off += sz
    @pl.when(s == 0)
    def _():
        pltpu.sync_copy(rowq.at[0].at[pl.ds(0, 16)], acc_sh.at[pl.ds(9984, 16)])
    plsc.subcore_barrier()

    pltpu.sync_copy(combo2.at[pl.ds(w * NCH2, NCH2)], comboall)

    def _derive(slot, ch):
        for g in range(8):
            v = comboall[ch, pl.ds(g * 16, 16)]
            d = v & 16383
            es = lax.shift_right_logical(v, 14)
            sr = es & 16383
            et = lax.shift_right_logical(es, 14)
            dstq[slot, pl.ds(g * 16, 16)] = d
            gixq[slot, pl.ds(g * 16, 16)] = et * NPAD + sr
            compq[slot, pl.ds(g * 16, 16)] = d * R + et

    def _fire(slot):
        pltpu.async_copy(rcp.at[compq.at[slot]], normq.at[slot], sn[slot])
        pltpu.async_copy(xw2.at[gixq.at[slot]], rowq.at[slot], sg[slot])

    for slot in (0, 1):
        _derive(slot, jnp.int32(slot))
        _fire(slot)

    def _pair(i, _):
        for slot in (0, 1):
            ch = 2 * i + slot
            pltpu.make_async_copy(xw2.at[gixq.at[slot]], rowq.at[slot],
                                  sg[slot]).wait()
            pltpu.make_async_copy(rcp.at[compq.at[slot]], normq.at[slot],
                                  sn[slot]).wait()
            def _scale(jj, _):
                j = 4 * jj
                for u in range(4):
                    nj = plsc.load_gather(
                        normq, [jnp.full((16,), slot, jnp.int32),
                                jnp.full((16,), j + u, jnp.int32)])
                    for k in range(4):
                        rowq[slot, j + u, pl.ds(k * 16, 16)] = (
                            rowq[slot, j + u, pl.ds(k * 16, 16)] * nj)
                return 0
            lax.fori_loop(0, RB // 4, _scale, 0)
            pltpu.sync_copy(rowq.at[slot], acc_sh.at[dstq.at[slot]], add=True)
            @pl.when(ch + 2 < NCH2)
            def _():
                _derive(slot, ch + 2)
                _fire(slot)
        return 0
    lax.fori_loop(0, NCH2 // 2, _pair, 0)
    plsc.subcore_barrier()

    base = s * 624
    @pl.when(c == 0)
    def _():
        pltpu.sync_copy(acc_sh.at[pl.ds(base, 624)], pA.at[pl.ds(base, 624)])
        @pl.when(s == 0)
        def _():
            pltpu.sync_copy(acc_sh.at[pl.ds(9984, 16)], pA.at[pl.ds(9984, 16)])
    @pl.when(c == 1)
    def _():
        pltpu.sync_copy(acc_sh.at[pl.ds(base, 624)], pB.at[pl.ds(base, 624)])
        @pl.when(s == 0)
        def _():
            pltpu.sync_copy(acc_sh.at[pl.ds(9984, 16)], pB.at[pl.ds(9984, 16)])


def _sc_agg2(xw2_f, combo2, rcp):
    return pl.kernel(
        _sc2_body,
        out_type=[
            jax.ShapeDtypeStruct((NPAD, C), jnp.float32),
            jax.ShapeDtypeStruct((NPAD, C), jnp.float32),
        ],
        mesh=plsc.VectorSubcoreMesh(**_SC_MESH),
        compiler_params=_SC_PARAMS2,
        scratch_types=[
            pltpu.VMEM_SHARED((N, C), jnp.float32),
            pltpu.VMEM((NCH2, RB), jnp.int32),
            pltpu.VMEM((2, RB, C), jnp.float32),
            pltpu.VMEM((2, RB), jnp.int32),
            pltpu.VMEM((2, RB), jnp.int32),
            pltpu.VMEM((2, RB), jnp.int32),
            pltpu.VMEM((2, RB), jnp.float32),
            pltpu.SemaphoreType.DMA,
            pltpu.SemaphoreType.DMA,
            pltpu.SemaphoreType.DMA,
            pltpu.SemaphoreType.DMA,
        ],
    )(xw2_f, combo2, rcp)


def _tc_c_body(p0_ref, p1_ref, root_ref, b2_ref, batch_ref,
               node_ref, graph_ref, acc_ref):
    i = pl.program_id(0)
    rid = i * BN + lax.broadcasted_iota(jnp.int32, (BN, 1), 0)
    nf = jnp.maximum(p0_ref[...] + p1_ref[...] + root_ref[...] + b2_ref[0],
                     0.0)
    nf = jnp.where(rid < N, nf, 0.0)
    m = jnp.max(nf, axis=1, keepdims=True)
    e = jnp.exp(nf - m)
    node_ref[...] = e / jnp.sum(e, axis=1, keepdims=True)

    bb = batch_ref[0]
    gids = lax.broadcasted_iota(jnp.int32, (G, BN), 0)
    oh = (bb == gids).astype(jnp.float32)
    ext = jnp.concatenate([nf, jnp.ones((BN, C), jnp.float32)], axis=1)
    contrib = jnp.dot(oh, ext, preferred_element_type=jnp.float32)
    @pl.when(i == 0)
    def _():
        acc_ref[...] = jnp.zeros_like(acc_ref)
    acc_ref[...] += contrib
    @pl.when(i == pl.num_programs(0) - 1)
    def _():
        rep = acc_ref[:, :C] / jnp.maximum(acc_ref[:, C:], 1.0)
        m2 = jnp.max(rep, axis=1, keepdims=True)
        e2 = jnp.exp(rep - m2)
        graph_ref[...] = e2 / jnp.sum(e2, axis=1, keepdims=True)


def _tc_c(pA, pB, root2, b2r, batch3d):
    return pl.pallas_call(
        _tc_c_body,
        grid=(NPAD // BN,),
        in_specs=[
            pl.BlockSpec((BN, C), lambda n: (n, 0)),
            pl.BlockSpec((BN, C), lambda n: (n, 0)),
            pl.BlockSpec((BN, C), lambda n: (n, 0)),
            pl.BlockSpec((1, C), lambda n: (0, 0)),
            pl.BlockSpec((1, 1, BN), lambda n: (n, 0, 0)),
        ],
        out_specs=[
            pl.BlockSpec((BN, C), lambda n: (n, 0)),
            pl.BlockSpec((G, G), lambda n: (0, 0)),
        ],
        out_shape=[
            jax.ShapeDtypeStruct((NPAD, C), jnp.float32),
            jax.ShapeDtypeStruct((G, G), jnp.float32),
        ],
        scratch_shapes=[pltpu.VMEM((G, 2 * C), jnp.float32)],
    )(pA, pB, root2, b2r, batch3d)


def kernel(x, edge_index, edge_type, batch, W1_rel, W1_root, b1,
           W2_rel, W2_root, b2):
    src = edge_index[0].astype(jnp.int32)
    dst = edge_index[1].astype(jnp.int32)
    et = edge_type.astype(jnp.int32)

    xpad = jnp.pad(x.astype(jnp.float32), ((0, NPAD - N), (0, 0)))
    w1cat = jnp.concatenate([W1_rel, W1_root[None]], axis=0)
    w2cat = jnp.concatenate([W2_rel, W2_root[None]], axis=0)

    pad_e = EPAD - E
    combo = (et * 16384 + src) * 16384 + dst
    combo2 = jnp.concatenate(
        [combo, jnp.full((pad_e,), N * 16384, jnp.int32)]).reshape(ECH, RB)
    comp2 = jnp.concatenate(
        [dst * R + et, jnp.full((pad_e,), N * R, jnp.int32)]).reshape(ECH, RB)
    batch3d = jnp.concatenate(
        [batch.astype(jnp.int32), jnp.full((NPAD - N,), G, jnp.int32)]
    ).reshape(NPAD // BN, 1, BN)

    rcp = _sc_count(comp2)[0]
    xwA, xwB = _tc_transform1(xpad.astype(jnp.bfloat16),
                              w1cat.astype(jnp.bfloat16))
    aggA, aggB = _sc_agg1(xwA, xwB, combo2, rcp)
    xw2 = _tc_b(aggA, aggB, xwA[8 * NPAD:], xwB[8 * NPAD:],
                b1.reshape(1, H), w2cat.astype(jnp.bfloat16))
    pA, pB = _sc_agg2(xw2, combo2, rcp)
    node_full, graph_out = _tc_c(pA, pB, xw2[8 * NPAD:], b2.reshape(1, C),
                                 batch3d)
    return (node_full[:N], graph_out)
